# baseline (speedup 1.0000x reference)
import jax
import jax.numpy as jnp
from jax import lax
from jax.experimental import pallas as pl
from jax.experimental.pallas import tpu as pltpu
from jax.experimental.pallas import tpu_sc as plsc

B = 1024
L = 200
D = 64
N = B * L
NW = 32
CH = 128
PER_W = N // NW
CHUNKS = PER_W // CH
V = 100000

_INV_2PI = 0.15915494309189535
_PI2 = 6.283185307179586
_RND = 12582912.0

_COS_C = (9.98971753e-01, -4.96206363e-01, 3.95066164e-02, -9.91486311e-04)
_SIN_C = (9.99880657e-01, -1.66227669e-01, 8.08460370e-03, -1.53090404e-04)


def _sc_body(g0_ref, g1_ref, poff_ref, t_ref, out_ref,
             ig0_v, ig1_v, po_v, g_v, out_v, sem_g, sem_o):
  wid = lax.axis_index("s") * 2 + lax.axis_index("c")
  base = wid * PER_W

  wsl = pl.ds(wid * CHUNKS, CHUNKS)
  pltpu.sync_copy(g0_ref.at[wsl], ig0_v)
  pltpu.sync_copy(g1_ref.at[wsl], ig1_v)
  pltpu.sync_copy(poff_ref.at[wsl], po_v)

  def gather_copies(c, nb):
    return (
        pltpu.make_async_copy(t_ref.at[ig0_v.at[c]], g_v.at[nb, 0],
                              sem_g.at[nb]),
        pltpu.make_async_copy(t_ref.at[ig1_v.at[c]], g_v.at[nb, 1],
                              sem_g.at[nb]),
    )

  def out_copy(c, nb):
    return pltpu.make_async_copy(
        out_v.at[nb], out_ref.at[pl.ds(base + c * CH, CH)], sem_o.at[nb])

  for cp in gather_copies(0, 0):
    cp.start()

  def do_chunk(c, carry):
    nb = c % 2

    @pl.when(c + 1 < CHUNKS)
    def _():
      for cp in gather_copies(c + 1, 1 - nb):
        cp.start()

    for cp in gather_copies(c, nb):
      cp.wait()

    @pl.when(c >= 2)
    def _():
      out_copy(c, nb).wait()

    pos0 = (c * CH) % L + 1

    @plsc.parallel_loop(0, CH // 16, carry=jnp.int32(pos0))
    def grp_body(g, pos_base):
      soffv = po_v[c, pl.ds(g * 16, 16)]
      for k in range(16):
        soff = soffv[k]
        fcol = soff + D
        fb = fcol >> 7
        fc = fcol & 127
        posk = pos_base + k
        posk = jnp.where(posk > L, posk - L, posk)
        posf = posk.astype(jnp.float32)
        i = g * 16 + k
        for j in range(D // 16):
          amp = g_v[nb, 0, i, pl.ds(soff + j * 16, 16)]
          f = g_v[nb, fb, i, pl.ds(fc + j * 16, 16)]
          bias = g_v[nb, 1, i, pl.ds(soff + j * 16, 16)]
          ph = posf * f + bias
          kf = (ph * _INV_2PI + _RND) - _RND
          r = ph - kf * _PI2
          u = r * r
          pc = jnp.float32(_COS_C[3])
          ps = jnp.float32(_SIN_C[3])
          for kk in range(2, -1, -1):
            pc = pc * u + jnp.float32(_COS_C[kk])
            ps = ps * u + jnp.float32(_SIN_C[kk])
          out_v[nb, i, pl.ds(j * 16, 16)] = amp * pc
          out_v[nb, i, pl.ds(D + j * 16, 16)] = (amp * r) * ps
      nxt = pos_base + 16
      return jnp.where(nxt > L, nxt - L, nxt)

    out_copy(c, nb).start()
    return carry

  lax.fori_loop(0, CHUNKS, do_chunk, 0)
  out_copy(CHUNKS - 2, 0).wait()
  out_copy(CHUNKS - 1, 1).wait()


@jax.jit
def _run(g0, g1, poff, fused, ):
  mesh = plsc.VectorSubcoreMesh(core_axis_name="c", subcore_axis_name="s")
  fn = pl.kernel(
      _sc_body,
      out_type=jax.ShapeDtypeStruct((N, 2 * D), jnp.float32),
      mesh=mesh,
      scratch_types=[
          pltpu.VMEM((CHUNKS, CH), jnp.int32),
          pltpu.VMEM((CHUNKS, CH), jnp.int32),
          pltpu.VMEM((CHUNKS, CH), jnp.int32),
          pltpu.VMEM((2, 2, CH, 2 * D), jnp.float32),
          pltpu.VMEM((2, CH, 2 * D), jnp.float32),
          pltpu.SemaphoreType.DMA((2,)),
          pltpu.SemaphoreType.DMA((2,)),
      ],
      compiler_params=pltpu.CompilerParams(use_tc_tiling_on_sc=False),
  )
  return fn(g0, g1, poff, fused)


def kernel(x, W_word, W_freq, W_phase):
  xf = x.reshape(N // CH, CH).astype(jnp.int32)
  g0 = (xf * 3) >> 1
  g1 = g0 + 1
  poff = (xf & 1) << 6
  fused = jnp.concatenate([W_word, W_freq, W_phase], axis=1)
  fused = fused.reshape(3 * V // 2, 2 * D)
  out = _run(g0, g1, poff, fused)
  return out.reshape(B, L, 2 * D)


# speedup vs baseline: 5.0568x; 5.0568x over previous
import jax
import jax.numpy as jnp
from jax import lax
from jax.experimental import pallas as pl
from jax.experimental.pallas import tpu as pltpu
from jax.experimental.pallas import tpu_sc as plsc

B = 1024
L = 200
D = 64
N = B * L
NW = 32
CH = 128
PER_W = N // NW
CHUNKS = PER_W // CH

_INV_2PI = 0.15915494309189535
_PI2 = 6.283185307179586
_RND = 12582912.0

_COS_C = (9.98971753e-01, -4.96206363e-01, 3.95066164e-02, -9.91486311e-04)
_SIN_C = (9.99880657e-01, -1.66227669e-01, 8.08460370e-03, -1.53090404e-04)


def _sc_body(x_ref, ww_ref, wf_ref, wp_ref, out_ref,
             idx_v, amp_v, freq_v, bias_v, out_v, sem_g, sem_o):
  wid = lax.axis_index("s") * 2 + lax.axis_index("c")
  base = wid * PER_W

  pltpu.sync_copy(x_ref.at[pl.ds(wid * CHUNKS, CHUNKS)], idx_v)

  def gather_copies(c, nb):
    isl = idx_v.at[c]
    return (pltpu.make_async_copy(ww_ref.at[isl], amp_v.at[nb], sem_g.at[nb]),
            pltpu.make_async_copy(wf_ref.at[isl], freq_v.at[nb], sem_g.at[nb]),
            pltpu.make_async_copy(wp_ref.at[isl], bias_v.at[nb], sem_g.at[nb]))

  def out_copy(c, nb):
    return pltpu.make_async_copy(
        out_v.at[nb], out_ref.at[pl.ds(base + c * CH, CH)], sem_o.at[nb])

  for cp in gather_copies(0, 0):
    cp.start()
  for cp in gather_copies(1, 1):
    cp.start()

  def do_chunk(c, nb):
    @pl.when(c + 2 < CHUNKS)
    def _():
      for cp in gather_copies(c + 2, (c + 2) % 3):
        cp.start()

    for cp in gather_copies(c, nb):
      cp.wait()

    nbo = c % 2

    @pl.when(c >= 2)
    def _():
      out_copy(c, nbo).wait()

    pos0 = (c * CH) % L + 1

    @plsc.parallel_loop(0, CH, carry=jnp.int32(pos0), unroll=2)
    def tok_body(i, pos):
      posf = pos.astype(jnp.float32)
      for j in range(D // 16):
        sl = pl.ds(j * 16, 16)
        f = freq_v[nb, i, sl]
        bias = bias_v[nb, i, sl]
        amp = amp_v[nb, i, sl]
        ph = posf * f + bias
        kf = (ph * _INV_2PI + _RND) - _RND
        r = ph - kf * _PI2
        u = r * r
        pc = jnp.float32(_COS_C[3])
        ps = jnp.float32(_SIN_C[3])
        for k in range(2, -1, -1):
          pc = pc * u + jnp.float32(_COS_C[k])
          ps = ps * u + jnp.float32(_SIN_C[k])
        out_v[nbo, i, sl] = amp * pc
        out_v[nbo, i, pl.ds(D + j * 16, 16)] = (amp * r) * ps
      return jnp.where(pos >= L, 1, pos + 1)

    out_copy(c, nbo).start()

  def chunk_body(c, carry):
    do_chunk(c, c % 3)
    return carry

  lax.fori_loop(0, CHUNKS, chunk_body, 0)
  out_copy(CHUNKS - 2, 0).wait()
  out_copy(CHUNKS - 1, 1).wait()


@jax.jit
def _run(x2d, W_word, W_freq, W_phase):
  mesh = plsc.VectorSubcoreMesh(core_axis_name="c", subcore_axis_name="s")
  fn = pl.kernel(
      _sc_body,
      out_type=jax.ShapeDtypeStruct((N, 2 * D), jnp.float32),
      mesh=mesh,
      scratch_types=[
          pltpu.VMEM((CHUNKS, CH), jnp.int32),
          pltpu.VMEM((3, CH, D), jnp.float32),
          pltpu.VMEM((3, CH, D), jnp.float32),
          pltpu.VMEM((3, CH, D), jnp.float32),
          pltpu.VMEM((2, CH, 2 * D), jnp.float32),
          pltpu.SemaphoreType.DMA((3,)),
          pltpu.SemaphoreType.DMA((2,)),
      ],
      compiler_params=pltpu.CompilerParams(use_tc_tiling_on_sc=False),
  )
  return fn(x2d, W_word, W_freq, W_phase)


def kernel(x, W_word, W_freq, W_phase):
  x2d = x.reshape(N // CH, CH).astype(jnp.int32)
  out = _run(x2d, W_word, W_freq, W_phase)
  return out.reshape(B, L, 2 * D)
